# Initial kernel scaffold; baseline (speedup 1.0000x reference)
import jax
import jax.numpy as jnp
from jax.experimental import pallas as pl


def kernel(pos, batch, params):
    raise NotImplementedError("write your pallas kernel here")


# speedup vs baseline: 1.1675x; 1.1675x over previous
import functools

import jax
import jax.numpy as jnp
from jax import lax
from jax.experimental import pallas as pl

B = 16
P = 1024
S1 = 512
S2 = 128
K = 64
NUM_CLASS = 10


def _fps_body(px, py, pz, n_pts, n_sample):
    iota_p = lax.broadcasted_iota(jnp.int32, (B, n_pts), 1)
    iota_s = lax.broadcasted_iota(jnp.int32, (B, n_sample), 1)

    selx0 = px[:, 0:1]
    sely0 = py[:, 0:1]
    selz0 = pz[:, 0:1]
    dists = (px - selx0) ** 2 + (py - sely0) ** 2 + (pz - selz0) ** 2

    idx_acc = jnp.zeros((B, n_sample), jnp.int32)
    p1x = jnp.where(iota_s == 0, selx0, 0.0)
    p1y = jnp.where(iota_s == 0, sely0, 0.0)
    p1z = jnp.where(iota_s == 0, selz0, 0.0)

    def body(i, carry):
        dists, idx_acc, p1x, p1y, p1z = carry
        m = jnp.max(dists, axis=1, keepdims=True)
        cand = jnp.where(dists == m, iota_p, n_pts * 2)
        nxt = jnp.min(cand, axis=1, keepdims=True)
        onehot = iota_p == nxt
        selx = jnp.sum(jnp.where(onehot, px, 0.0), axis=1, keepdims=True)
        sely = jnp.sum(jnp.where(onehot, py, 0.0), axis=1, keepdims=True)
        selz = jnp.sum(jnp.where(onehot, pz, 0.0), axis=1, keepdims=True)
        d = (px - selx) ** 2 + (py - sely) ** 2 + (pz - selz) ** 2
        dists = jnp.minimum(dists, d)
        here = iota_s == i
        idx_acc = jnp.where(here, nxt, idx_acc)
        p1x = jnp.where(here, selx, p1x)
        p1y = jnp.where(here, sely, p1y)
        p1z = jnp.where(here, selz, p1z)
        return dists, idx_acc, p1x, p1y, p1z

    carry = (dists, idx_acc, p1x, p1y, p1z)
    carry = lax.fori_loop(1, n_sample, body, carry)
    _, idx_acc, p1x, p1y, p1z = carry
    return idx_acc, p1x, p1y, p1z


def _fps_kernel(px_ref, py_ref, pz_ref,
                idx1_ref, p1x_ref, p1y_ref, p1z_ref,
                idx2_ref, p2x_ref, p2y_ref, p2z_ref):
    px = px_ref[...]
    py = py_ref[...]
    pz = pz_ref[...]
    idx1, p1x, p1y, p1z = _fps_body(px, py, pz, P, S1)
    idx1_ref[...] = idx1
    p1x_ref[...] = p1x
    p1y_ref[...] = p1y
    p1z_ref[...] = p1z
    idx2, p2x, p2y, p2z = _fps_body(p1x, p1y, p1z, S1, S2)
    idx2_ref[...] = idx2
    p2x_ref[...] = p2x
    p2y_ref[...] = p2y
    p2z_ref[...] = p2z


def _run_fps(px, py, pz):
    out_shape = (
        jax.ShapeDtypeStruct((B, S1), jnp.int32),
        jax.ShapeDtypeStruct((B, S1), jnp.float32),
        jax.ShapeDtypeStruct((B, S1), jnp.float32),
        jax.ShapeDtypeStruct((B, S1), jnp.float32),
        jax.ShapeDtypeStruct((B, S2), jnp.int32),
        jax.ShapeDtypeStruct((B, S2), jnp.float32),
        jax.ShapeDtypeStruct((B, S2), jnp.float32),
        jax.ShapeDtypeStruct((B, S2), jnp.float32),
    )
    return pl.pallas_call(_fps_kernel, out_shape=out_shape)(px, py, pz)


_ROWS_BLK = 4096


def _pc1_kernel(rel_ref, valid_ref, w1_ref, b1_ref, w2_ref, b2_ref,
                w3_ref, b3_ref, out_ref):
    h = jnp.dot(rel_ref[...], w1_ref[...], preferred_element_type=jnp.float32)
    h = jnp.maximum(h + b1_ref[...], 0.0)
    h = jnp.dot(h, w2_ref[...], preferred_element_type=jnp.float32)
    h = jnp.maximum(h + b2_ref[...], 0.0)
    h = jnp.dot(h, w3_ref[...], preferred_element_type=jnp.float32)
    h = h + b3_ref[...]
    h = jnp.where(valid_ref[...] > 0, h, -jnp.inf)
    out_ref[...] = jnp.max(h.reshape(_ROWS_BLK // K, K, h.shape[-1]), axis=1)


def _run_pc1(rel, valid, layers):
    (w1, b1), (w2, b2), (w3, b3) = layers
    n = rel.shape[0]
    grid = n // _ROWS_BLK
    qblk = _ROWS_BLK // K
    co = w3.shape[1]
    full = lambda a: pl.BlockSpec(a.shape, lambda i: (0,) * a.ndim)
    return pl.pallas_call(
        _pc1_kernel,
        grid=(grid,),
        in_specs=[
            pl.BlockSpec((_ROWS_BLK, 3), lambda i: (i, 0)),
            pl.BlockSpec((_ROWS_BLK, 1), lambda i: (i, 0)),
            full(w1), full(b1.reshape(1, -1)),
            full(w2), full(b2.reshape(1, -1)),
            full(w3), full(b3.reshape(1, -1)),
        ],
        out_specs=pl.BlockSpec((qblk, co), lambda i: (i, 0)),
        out_shape=jax.ShapeDtypeStruct((n // K, co), jnp.float32),
    )(rel, valid, w1, b1.reshape(1, -1), w2, b2.reshape(1, -1),
      w3, b3.reshape(1, -1))


def _pc2_kernel(xj_ref, rel_ref, valid_ref, w1a_ref, w1b_ref, b1_ref,
                w2_ref, b2_ref, w3_ref, b3_ref, out_ref):
    h = jnp.dot(xj_ref[...], w1a_ref[...], preferred_element_type=jnp.float32)
    h = h + jnp.dot(rel_ref[...], w1b_ref[...],
                    preferred_element_type=jnp.float32)
    h = jnp.maximum(h + b1_ref[...], 0.0)
    h = jnp.dot(h, w2_ref[...], preferred_element_type=jnp.float32)
    h = jnp.maximum(h + b2_ref[...], 0.0)
    h = jnp.dot(h, w3_ref[...], preferred_element_type=jnp.float32)
    h = h + b3_ref[...]
    h = jnp.where(valid_ref[...] > 0, h, -jnp.inf)
    out_ref[...] = jnp.max(h.reshape(_ROWS_BLK // K, K, h.shape[-1]), axis=1)


def _run_pc2(xj, rel, valid, layers):
    (w1, b1), (w2, b2), (w3, b3) = layers
    ci = xj.shape[1]
    w1a, w1b = w1[:ci], w1[ci:]
    n = xj.shape[0]
    grid = n // _ROWS_BLK
    qblk = _ROWS_BLK // K
    co = w3.shape[1]
    full = lambda a: pl.BlockSpec(a.shape, lambda i: (0,) * a.ndim)
    return pl.pallas_call(
        _pc2_kernel,
        grid=(grid,),
        in_specs=[
            pl.BlockSpec((_ROWS_BLK, ci), lambda i: (i, 0)),
            pl.BlockSpec((_ROWS_BLK, 3), lambda i: (i, 0)),
            pl.BlockSpec((_ROWS_BLK, 1), lambda i: (i, 0)),
            full(w1a), full(w1b), full(b1.reshape(1, -1)),
            full(w2), full(b2.reshape(1, -1)),
            full(w3), full(b3.reshape(1, -1)),
        ],
        out_specs=pl.BlockSpec((qblk, co), lambda i: (i, 0)),
        out_shape=jax.ShapeDtypeStruct((n // K, co), jnp.float32),
    )(xj, rel, valid, w1a, w1b, b1.reshape(1, -1), w2, b2.reshape(1, -1),
      w3, b3.reshape(1, -1))


def _glob_kernel(feat_ref, w1_ref, b1_ref, w2_ref, b2_ref, w3_ref, b3_ref,
                 out_ref):
    h = jnp.dot(feat_ref[...], w1_ref[...], preferred_element_type=jnp.float32)
    h = jnp.maximum(h + b1_ref[...], 0.0)
    h = jnp.dot(h, w2_ref[...], preferred_element_type=jnp.float32)
    h = jnp.maximum(h + b2_ref[...], 0.0)
    h = jnp.dot(h, w3_ref[...], preferred_element_type=jnp.float32)
    h = h + b3_ref[...]
    out_ref[...] = jnp.max(h, axis=0, keepdims=True)[None]


def _run_glob(feat, layers):
    (w1, b1), (w2, b2), (w3, b3) = layers
    ci = feat.shape[1]
    co = w3.shape[1]
    full = lambda a: pl.BlockSpec(a.shape, lambda i: (0,) * a.ndim)
    return pl.pallas_call(
        _glob_kernel,
        grid=(B,),
        in_specs=[
            pl.BlockSpec((S2, ci), lambda i: (i, 0)),
            full(w1), full(b1.reshape(1, -1)),
            full(w2), full(b2.reshape(1, -1)),
            full(w3), full(b3.reshape(1, -1)),
        ],
        out_specs=pl.BlockSpec((1, 1, co), lambda i: (i, 0, 0)),
        out_shape=jax.ShapeDtypeStruct((B, 1, co), jnp.float32),
    )(feat, w1, b1.reshape(1, -1), w2, b2.reshape(1, -1), w3,
      b3.reshape(1, -1)).reshape(B, co)


def _head_kernel(g_ref, w1_ref, b1_ref, w2_ref, b2_ref, out_ref):
    h = jnp.dot(g_ref[...], w1_ref[...], preferred_element_type=jnp.float32)
    h = jnp.maximum(h + b1_ref[...], 0.0)
    h = jnp.dot(h, w2_ref[...], preferred_element_type=jnp.float32)
    h = h + b2_ref[...]
    m = jnp.max(h, axis=1, keepdims=True)
    e = jnp.exp(h - m)
    out_ref[...] = (h - m) - jnp.log(jnp.sum(e, axis=1, keepdims=True))


def _run_head(g, layers):
    (w1, b1), (w2, b2) = layers
    return pl.pallas_call(
        _head_kernel,
        out_shape=jax.ShapeDtypeStruct((B, NUM_CLASS), jnp.float32),
    )(g, w1, b1.reshape(1, -1), w2, b2.reshape(1, -1))


def _group(pos, pos_q, r, k):
    d = jnp.sum((pos_q[:, :, None, :] - pos[:, None, :, :]) ** 2, axis=-1)
    d = jnp.where(d <= r * r, d, jnp.inf)
    neg_vals, nbr = lax.top_k(-d, k)
    valid = neg_vals > -jnp.inf
    return nbr, valid


def kernel(pos, batch, params):
    del batch
    pos = pos.reshape(B, P, 3)
    px, py, pz = pos[:, :, 0], pos[:, :, 1], pos[:, :, 2]
    (idx1, p1x, p1y, p1z, idx2, p2x, p2y, p2z) = _run_fps(px, py, pz)
    pos1 = jnp.stack([p1x, p1y, p1z], axis=-1)
    pos2 = jnp.stack([p2x, p2y, p2z], axis=-1)
    ar = jnp.arange(B)

    nbr1, valid1 = _group(pos, pos1, 0.2, K)
    pos_j = pos[ar[:, None, None], nbr1]
    rel1 = (pos_j - pos1[:, :, None, :]).reshape(B * S1 * K, 3)
    v1 = valid1.reshape(B * S1 * K, 1).astype(jnp.float32)
    x1 = _run_pc1(rel1, v1, params['sa1'])
    x1 = x1.reshape(B, S1, 128)

    nbr2, valid2 = _group(pos1, pos2, 0.4, K)
    pos_j2 = pos1[ar[:, None, None], nbr2]
    rel2 = (pos_j2 - pos2[:, :, None, :]).reshape(B * S2 * K, 3)
    xj2 = x1[ar[:, None, None], nbr2].reshape(B * S2 * K, 128)
    v2 = valid2.reshape(B * S2 * K, 1).astype(jnp.float32)
    x2 = _run_pc2(xj2, rel2, v2, params['sa2'])

    feat = jnp.concatenate([x2, pos2.reshape(B * S2, 3)], axis=-1)
    g = _run_glob(feat, params['sa3'])
    return _run_head(g, params['head'])


# speedup vs baseline: 13.5556x; 11.6105x over previous
import functools

import jax
import jax.numpy as jnp
from jax import lax
from jax.experimental import pallas as pl
from jax.experimental.pallas import tpu as pltpu
from jax.experimental.pallas import tpu_sc as plsc

B = 16
P = 1024
S1 = 512
S2 = 128
K = 64
NUM_CLASS = 10

SC_NC = 2
SC_NS = 16
SC_NW = SC_NC * SC_NS
SC_L = 16


def _fps_body(px, py, pz, n_pts, n_sample):
    iota_p = lax.broadcasted_iota(jnp.int32, (B, n_pts), 1)
    iota_s = lax.broadcasted_iota(jnp.int32, (B, n_sample), 1)

    selx0 = px[:, 0:1]
    sely0 = py[:, 0:1]
    selz0 = pz[:, 0:1]
    dists = (px - selx0) ** 2 + (py - sely0) ** 2 + (pz - selz0) ** 2

    idx_acc = jnp.zeros((B, n_sample), jnp.int32)
    p1x = jnp.where(iota_s == 0, selx0, 0.0)
    p1y = jnp.where(iota_s == 0, sely0, 0.0)
    p1z = jnp.where(iota_s == 0, selz0, 0.0)

    def body(i, carry):
        dists, idx_acc, p1x, p1y, p1z = carry
        m = jnp.max(dists, axis=1, keepdims=True)
        cand = jnp.where(dists == m, iota_p, n_pts * 2)
        nxt = jnp.min(cand, axis=1, keepdims=True)
        onehot = iota_p == nxt
        selx = jnp.sum(jnp.where(onehot, px, 0.0), axis=1, keepdims=True)
        sely = jnp.sum(jnp.where(onehot, py, 0.0), axis=1, keepdims=True)
        selz = jnp.sum(jnp.where(onehot, pz, 0.0), axis=1, keepdims=True)
        d = (px - selx) ** 2 + (py - sely) ** 2 + (pz - selz) ** 2
        dists = jnp.minimum(dists, d)
        here = iota_s == i
        idx_acc = jnp.where(here, nxt, idx_acc)
        p1x = jnp.where(here, selx, p1x)
        p1y = jnp.where(here, sely, p1y)
        p1z = jnp.where(here, selz, p1z)
        return dists, idx_acc, p1x, p1y, p1z

    carry = (dists, idx_acc, p1x, p1y, p1z)
    carry = lax.fori_loop(1, n_sample, body, carry)
    _, idx_acc, p1x, p1y, p1z = carry
    return idx_acc, p1x, p1y, p1z


def _fps_kernel(px_ref, py_ref, pz_ref,
                idx1_ref, p1x_ref, p1y_ref, p1z_ref,
                idx2_ref, p2x_ref, p2y_ref, p2z_ref):
    px = px_ref[...]
    py = py_ref[...]
    pz = pz_ref[...]
    idx1, p1x, p1y, p1z = _fps_body(px, py, pz, P, S1)
    idx1_ref[...] = idx1
    p1x_ref[...] = p1x
    p1y_ref[...] = p1y
    p1z_ref[...] = p1z
    idx2, p2x, p2y, p2z = _fps_body(p1x, p1y, p1z, S1, S2)
    idx2_ref[...] = idx2
    p2x_ref[...] = p2x
    p2y_ref[...] = p2y
    p2z_ref[...] = p2z


def _run_fps(px, py, pz):
    out_shape = (
        jax.ShapeDtypeStruct((B, S1), jnp.int32),
        jax.ShapeDtypeStruct((B, S1), jnp.float32),
        jax.ShapeDtypeStruct((B, S1), jnp.float32),
        jax.ShapeDtypeStruct((B, S1), jnp.float32),
        jax.ShapeDtypeStruct((B, S2), jnp.int32),
        jax.ShapeDtypeStruct((B, S2), jnp.float32),
        jax.ShapeDtypeStruct((B, S2), jnp.float32),
        jax.ShapeDtypeStruct((B, S2), jnp.float32),
    )
    return pl.pallas_call(_fps_kernel, out_shape=out_shape)(px, py, pz)


_ROWS_BLK = 4096


def _pc1_kernel(rel_ref, w1_ref, b1_ref, w2_ref, b2_ref,
                w3_ref, b3_ref, out_ref):
    h = jnp.dot(rel_ref[...], w1_ref[...], preferred_element_type=jnp.float32)
    h = jnp.maximum(h + b1_ref[...], 0.0)
    h = jnp.dot(h, w2_ref[...], preferred_element_type=jnp.float32)
    h = jnp.maximum(h + b2_ref[...], 0.0)
    h = jnp.dot(h, w3_ref[...], preferred_element_type=jnp.float32)
    h = h + b3_ref[...]
    out_ref[...] = jnp.max(h.reshape(_ROWS_BLK // K, K, h.shape[-1]), axis=1)


def _run_pc1(rel, layers):
    (w1, b1), (w2, b2), (w3, b3) = layers
    n = rel.shape[0]
    grid = n // _ROWS_BLK
    qblk = _ROWS_BLK // K
    co = w3.shape[1]
    full = lambda a: pl.BlockSpec(a.shape, lambda i: (0,) * a.ndim)
    return pl.pallas_call(
        _pc1_kernel,
        grid=(grid,),
        in_specs=[
            pl.BlockSpec((_ROWS_BLK, 3), lambda i: (i, 0)),
            full(w1), full(b1.reshape(1, -1)),
            full(w2), full(b2.reshape(1, -1)),
            full(w3), full(b3.reshape(1, -1)),
        ],
        out_specs=pl.BlockSpec((qblk, co), lambda i: (i, 0)),
        out_shape=jax.ShapeDtypeStruct((n // K, co), jnp.float32),
    )(rel, w1, b1.reshape(1, -1), w2, b2.reshape(1, -1),
      w3, b3.reshape(1, -1))


def _pc2_kernel(xj_ref, rel_ref, w1a_ref, w1b_ref, b1_ref,
                w2_ref, b2_ref, w3_ref, b3_ref, out_ref):
    h = jnp.dot(xj_ref[...], w1a_ref[...], preferred_element_type=jnp.float32)
    h = h + jnp.dot(rel_ref[...], w1b_ref[...],
                    preferred_element_type=jnp.float32)
    h = jnp.maximum(h + b1_ref[...], 0.0)
    h = jnp.dot(h, w2_ref[...], preferred_element_type=jnp.float32)
    h = jnp.maximum(h + b2_ref[...], 0.0)
    h = jnp.dot(h, w3_ref[...], preferred_element_type=jnp.float32)
    h = h + b3_ref[...]
    out_ref[...] = jnp.max(h.reshape(_ROWS_BLK // K, K, h.shape[-1]), axis=1)


def _run_pc2(xj, rel, layers):
    (w1, b1), (w2, b2), (w3, b3) = layers
    ci = xj.shape[1]
    w1a, w1b = w1[:ci], w1[ci:]
    n = xj.shape[0]
    grid = n // _ROWS_BLK
    qblk = _ROWS_BLK // K
    co = w3.shape[1]
    full = lambda a: pl.BlockSpec(a.shape, lambda i: (0,) * a.ndim)
    return pl.pallas_call(
        _pc2_kernel,
        grid=(grid,),
        in_specs=[
            pl.BlockSpec((_ROWS_BLK, ci), lambda i: (i, 0)),
            pl.BlockSpec((_ROWS_BLK, 3), lambda i: (i, 0)),
            full(w1a), full(w1b), full(b1.reshape(1, -1)),
            full(w2), full(b2.reshape(1, -1)),
            full(w3), full(b3.reshape(1, -1)),
        ],
        out_specs=pl.BlockSpec((qblk, co), lambda i: (i, 0)),
        out_shape=jax.ShapeDtypeStruct((n // K, co), jnp.float32),
    )(xj, rel, w1a, w1b, b1.reshape(1, -1), w2, b2.reshape(1, -1),
      w3, b3.reshape(1, -1))


def _glob_kernel(feat_ref, w1_ref, b1_ref, w2_ref, b2_ref, w3_ref, b3_ref,
                 out_ref):
    h = jnp.dot(feat_ref[...], w1_ref[...], preferred_element_type=jnp.float32)
    h = jnp.maximum(h + b1_ref[...], 0.0)
    h = jnp.dot(h, w2_ref[...], preferred_element_type=jnp.float32)
    h = jnp.maximum(h + b2_ref[...], 0.0)
    h = jnp.dot(h, w3_ref[...], preferred_element_type=jnp.float32)
    h = h + b3_ref[...]
    out_ref[...] = jnp.max(h, axis=0, keepdims=True)[None]


def _run_glob(feat, layers):
    (w1, b1), (w2, b2), (w3, b3) = layers
    ci = feat.shape[1]
    co = w3.shape[1]
    full = lambda a: pl.BlockSpec(a.shape, lambda i: (0,) * a.ndim)
    return pl.pallas_call(
        _glob_kernel,
        grid=(B,),
        in_specs=[
            pl.BlockSpec((S2, ci), lambda i: (i, 0)),
            full(w1), full(b1.reshape(1, -1)),
            full(w2), full(b2.reshape(1, -1)),
            full(w3), full(b3.reshape(1, -1)),
        ],
        out_specs=pl.BlockSpec((1, 1, co), lambda i: (i, 0, 0)),
        out_shape=jax.ShapeDtypeStruct((B, 1, co), jnp.float32),
    )(feat, w1, b1.reshape(1, -1), w2, b2.reshape(1, -1), w3,
      b3.reshape(1, -1)).reshape(B, co)


def _head_kernel(g_ref, w1_ref, b1_ref, w2_ref, b2_ref, out_ref):
    h = jnp.dot(g_ref[...], w1_ref[...], preferred_element_type=jnp.float32)
    h = jnp.maximum(h + b1_ref[...], 0.0)
    h = jnp.dot(h, w2_ref[...], preferred_element_type=jnp.float32)
    h = h + b2_ref[...]
    m = jnp.max(h, axis=1, keepdims=True)
    e = jnp.exp(h - m)
    out_ref[...] = (h - m) - jnp.log(jnp.sum(e, axis=1, keepdims=True))


def _run_head(g, layers):
    (w1, b1), (w2, b2) = layers
    return pl.pallas_call(
        _head_kernel,
        out_shape=jax.ShapeDtypeStruct((B, NUM_CLASS), jnp.float32),
    )(g, w1, b1.reshape(1, -1), w2, b2.reshape(1, -1))


_BUF = K + 2 * SC_L


def _search_row(pxv, pyv, pzv, bufv, qxs, qys, qzs, selfs, rr, n_chunks,
                iota16):
    trash = K + SC_L + iota16
    for s in range(_BUF // SC_L):
        bufv[pl.ds(s * SC_L, SC_L)] = selfs

    def chunk(c, cnt):
        base = c * SC_L
        dx = pxv[pl.ds(base, SC_L)] - qxs
        dy = pyv[pl.ds(base, SC_L)] - qys
        dz = pzv[pl.ds(base, SC_L)] - qzs
        dsq = dx * dx + dy * dy + dz * dz
        mask = dsq <= rr
        mi = mask.astype(jnp.int32)
        cums = plsc.cumsum(mi)
        slots = jnp.where(mask, cnt + cums - mi, trash)
        plsc.store_scatter(bufv, [slots], iota16 + base)
        return jnp.minimum(cnt + cums[SC_L - 1], K)

    lax.fori_loop(0, n_chunks, chunk, 0)


_QW1 = S1 // 2


def _rs1_kernel(px_hbm, py_hbm, pz_hbm, qx_hbm, qy_hbm, qz_hbm, self_hbm,
                rx_hbm, ry_hbm, rz_hbm,
                pxv, pyv, pzv, qxv, qyv, qzv, selfv, bufv, rxv, ryv, rzv):
    wid = lax.axis_index("s") * SC_NC + lax.axis_index("c")
    b = wid // 2
    h = wid % 2
    pltpu.sync_copy(px_hbm.at[b], pxv)
    pltpu.sync_copy(py_hbm.at[b], pyv)
    pltpu.sync_copy(pz_hbm.at[b], pzv)
    q0 = h * _QW1
    pltpu.sync_copy(qx_hbm.at[b, pl.ds(q0, _QW1)], qxv)
    pltpu.sync_copy(qy_hbm.at[b, pl.ds(q0, _QW1)], qyv)
    pltpu.sync_copy(qz_hbm.at[b, pl.ds(q0, _QW1)], qzv)
    pltpu.sync_copy(self_hbm.at[b, pl.ds(q0, _QW1)], selfv)
    iota16 = lax.broadcasted_iota(jnp.int32, (SC_L,), 0)
    rr = jnp.float32(0.2 * 0.2)

    def qchunk(qb, _):
        qx16 = qxv[pl.ds(qb * SC_L, SC_L)]
        qy16 = qyv[pl.ds(qb * SC_L, SC_L)]
        qz16 = qzv[pl.ds(qb * SC_L, SC_L)]
        self16 = selfv[pl.ds(qb * SC_L, SC_L)]
        for j in range(SC_L):
            qi = qb * SC_L + j
            qxs = jnp.full((SC_L,), qx16[j], jnp.float32)
            qys = jnp.full((SC_L,), qy16[j], jnp.float32)
            qzs = jnp.full((SC_L,), qz16[j], jnp.float32)
            selfs = jnp.full((SC_L,), self16[j], jnp.int32)
            _search_row(pxv, pyv, pzv, bufv, qxs, qys, qzs, selfs, rr,
                        P // SC_L, iota16)
            for s in range(K // SC_L):
                idxv = bufv[pl.ds(s * SC_L, SC_L)]
                rxv[qi, pl.ds(s * SC_L, SC_L)] = (
                    plsc.load_gather(pxv, [idxv]) - qxs)
                ryv[qi, pl.ds(s * SC_L, SC_L)] = (
                    plsc.load_gather(pyv, [idxv]) - qys)
                rzv[qi, pl.ds(s * SC_L, SC_L)] = (
                    plsc.load_gather(pzv, [idxv]) - qzs)
        return 0

    lax.fori_loop(0, _QW1 // SC_L, qchunk, 0)
    pltpu.sync_copy(rxv, rx_hbm.at[b, pl.ds(q0, _QW1)])
    pltpu.sync_copy(ryv, ry_hbm.at[b, pl.ds(q0, _QW1)])
    pltpu.sync_copy(rzv, rz_hbm.at[b, pl.ds(q0, _QW1)])


def _run_rs1(px, py, pz, qx, qy, qz, self_idx):
    mesh = plsc.VectorSubcoreMesh(core_axis_name="c", subcore_axis_name="s",
                                  num_cores=SC_NC, num_subcores=SC_NS)
    f32 = jnp.float32
    out_type = tuple(jax.ShapeDtypeStruct((B, S1, K), f32) for _ in range(3))
    fn = pl.kernel(
        _rs1_kernel,
        out_type=out_type,
        mesh=mesh,
        scratch_types=[
            pltpu.VMEM((P,), f32), pltpu.VMEM((P,), f32),
            pltpu.VMEM((P,), f32),
            pltpu.VMEM((_QW1,), f32), pltpu.VMEM((_QW1,), f32),
            pltpu.VMEM((_QW1,), f32),
            pltpu.VMEM((_QW1,), jnp.int32),
            pltpu.VMEM((_BUF,), jnp.int32),
            pltpu.VMEM((_QW1, K), f32), pltpu.VMEM((_QW1, K), f32),
            pltpu.VMEM((_QW1, K), f32),
        ],
        compiler_params=pltpu.CompilerParams(needs_layout_passes=False),
    )
    return fn(px, py, pz, qx, qy, qz, self_idx)


_QW2 = S2 // 2
_GRP = 8


def _rs2_kernel(px_hbm, py_hbm, pz_hbm, qx_hbm, qy_hbm, qz_hbm, self_hbm,
                x1_hbm,
                rx_hbm, ry_hbm, rz_hbm, xj_hbm,
                pxv, pyv, pzv, qxv, qyv, qzv, selfv, bufv, rxv, ryv, rzv,
                idxg, rows_v, sem):
    wid = lax.axis_index("s") * SC_NC + lax.axis_index("c")
    b = wid // 2
    h = wid % 2
    pltpu.sync_copy(px_hbm.at[b], pxv)
    pltpu.sync_copy(py_hbm.at[b], pyv)
    pltpu.sync_copy(pz_hbm.at[b], pzv)
    q0 = h * _QW2
    pltpu.sync_copy(qx_hbm.at[b, pl.ds(q0, _QW2)], qxv)
    pltpu.sync_copy(qy_hbm.at[b, pl.ds(q0, _QW2)], qyv)
    pltpu.sync_copy(qz_hbm.at[b, pl.ds(q0, _QW2)], qzv)
    pltpu.sync_copy(self_hbm.at[b, pl.ds(q0, _QW2)], selfv)
    iota16 = lax.broadcasted_iota(jnp.int32, (SC_L,), 0)
    rr = jnp.float32(0.4 * 0.4)
    row_base = jnp.int32(b * S2 + q0)

    def qchunk(qb, _):
        qx16 = qxv[pl.ds(qb * SC_L, SC_L)]
        qy16 = qyv[pl.ds(qb * SC_L, SC_L)]
        qz16 = qzv[pl.ds(qb * SC_L, SC_L)]
        self16 = selfv[pl.ds(qb * SC_L, SC_L)]
        for half in range(SC_L // _GRP):
            for j in range(_GRP):
                lane = half * _GRP + j
                qi = qb * SC_L + lane
                qxs = jnp.full((SC_L,), qx16[lane], jnp.float32)
                qys = jnp.full((SC_L,), qy16[lane], jnp.float32)
                qzs = jnp.full((SC_L,), qz16[lane], jnp.float32)
                selfs = jnp.full((SC_L,), self16[lane], jnp.int32)
                _search_row(pxv, pyv, pzv, bufv, qxs, qys, qzs, selfs, rr,
                            S1 // SC_L, iota16)
                for s in range(K // SC_L):
                    idxv = bufv[pl.ds(s * SC_L, SC_L)]
                    rxv[qi, pl.ds(s * SC_L, SC_L)] = (
                        plsc.load_gather(pxv, [idxv]) - qxs)
                    ryv[qi, pl.ds(s * SC_L, SC_L)] = (
                        plsc.load_gather(pyv, [idxv]) - qys)
                    rzv[qi, pl.ds(s * SC_L, SC_L)] = (
                        plsc.load_gather(pzv, [idxv]) - qzs)
                    idxg[pl.ds(j * K + s * SC_L, SC_L)] = idxv + b * S1
            pltpu.async_copy(x1_hbm.at[idxg], rows_v, sem).wait()
            row0 = row_base + qb * SC_L + half * _GRP
            pltpu.sync_copy(rows_v, xj_hbm.at[pl.ds(row0 * K, _GRP * K)])
        return 0

    lax.fori_loop(0, _QW2 // SC_L, qchunk, 0)
    pltpu.sync_copy(rxv, rx_hbm.at[b, pl.ds(q0, _QW2)])
    pltpu.sync_copy(ryv, ry_hbm.at[b, pl.ds(q0, _QW2)])
    pltpu.sync_copy(rzv, rz_hbm.at[b, pl.ds(q0, _QW2)])


def _run_rs2(px, py, pz, qx, qy, qz, self_idx, x1):
    mesh = plsc.VectorSubcoreMesh(core_axis_name="c", subcore_axis_name="s",
                                  num_cores=SC_NC, num_subcores=SC_NS)
    f32 = jnp.float32
    out_type = (
        jax.ShapeDtypeStruct((B, S2, K), f32),
        jax.ShapeDtypeStruct((B, S2, K), f32),
        jax.ShapeDtypeStruct((B, S2, K), f32),
        jax.ShapeDtypeStruct((B * S2 * K, 128), f32),
    )
    fn = pl.kernel(
        _rs2_kernel,
        out_type=out_type,
        mesh=mesh,
        scratch_types=[
            pltpu.VMEM((S1,), f32), pltpu.VMEM((S1,), f32),
            pltpu.VMEM((S1,), f32),
            pltpu.VMEM((_QW2,), f32), pltpu.VMEM((_QW2,), f32),
            pltpu.VMEM((_QW2,), f32),
            pltpu.VMEM((_QW2,), jnp.int32),
            pltpu.VMEM((_BUF,), jnp.int32),
            pltpu.VMEM((_QW2, K), f32), pltpu.VMEM((_QW2, K), f32),
            pltpu.VMEM((_QW2, K), f32),
            pltpu.VMEM((_GRP * K,), jnp.int32),
            pltpu.VMEM((_GRP * K, 128), f32),
            pltpu.SemaphoreType.DMA,
        ],
        compiler_params=pltpu.CompilerParams(needs_layout_passes=False),
    )
    return fn(px, py, pz, qx, qy, qz, self_idx, x1)


def kernel(pos, batch, params):
    del batch
    pos = pos.reshape(B, P, 3)
    px, py, pz = pos[:, :, 0], pos[:, :, 1], pos[:, :, 2]
    (idx1, p1x, p1y, p1z, idx2, p2x, p2y, p2z) = _run_fps(px, py, pz)

    rx1, ry1, rz1 = _run_rs1(px, py, pz, p1x, p1y, p1z, idx1)
    rel1 = jnp.stack(
        [rx1.reshape(-1), ry1.reshape(-1), rz1.reshape(-1)], axis=-1)
    x1 = _run_pc1(rel1, params['sa1'])

    rx2, ry2, rz2, xj2 = _run_rs2(p1x, p1y, p1z, p2x, p2y, p2z, idx2, x1)
    rel2 = jnp.stack(
        [rx2.reshape(-1), ry2.reshape(-1), rz2.reshape(-1)], axis=-1)
    x2 = _run_pc2(xj2, rel2, params['sa2'])

    pos2 = jnp.stack([p2x, p2y, p2z], axis=-1)
    feat = jnp.concatenate([x2, pos2.reshape(B * S2, 3)], axis=-1)
    g = _run_glob(feat, params['sa3'])
    return _run_head(g, params['head'])
